# baseline (speedup 1.0000x reference)
import functools
import math

import jax
import jax.numpy as jnp
from jax import lax
from jax.experimental import pallas as pl
from jax.experimental.pallas import tpu as pltpu
from jax.experimental.pallas import tpu_sc as plsc

D = 64
SEQ = 50
SUB = 100
CHUNK = 2 * SUB
NBUF = 4


def _sc_geometry():
    try:
        info = plsc.get_sparse_core_info()
        return info.num_cores, info.num_subcores
    except Exception:
        return 2, 16


def _build_sc_kernel(n_rows, nc, ns):
    nw = nc * ns
    rpw = n_rows // nw
    n_chunks = rpw // CHUNK
    subs_per_w = rpw // SUB
    n_rounds = n_chunks // NBUF

    mesh = plsc.VectorSubcoreMesh(core_axis_name="c", subcore_axis_name="s")

    @functools.partial(
        pl.kernel,
        out_type=jax.ShapeDtypeStruct((n_rows, D), jnp.float32),
        mesh=mesh,
        scratch_types=[
            pltpu.VMEM((subs_per_w, SUB), jnp.int32),
            pltpu.VMEM((SEQ, D), jnp.float32),
        ] + [pltpu.VMEM((CHUNK, D), jnp.float32) for _ in range(NBUF)]
          + [pltpu.SemaphoreType.DMA for _ in range(2 * NBUF)],
        compiler_params=pltpu.CompilerParams(use_tc_tiling_on_sc=False),
    )
    def k(idx_hbm, table_hbm, pe_hbm, out_hbm, idx_v, pe_v, *bufs_and_sems):
        bufs = bufs_and_sems[:NBUF]
        sem_g = bufs_and_sems[NBUF:2 * NBUF]
        sem_w = bufs_and_sems[2 * NBUF:]

        cid = lax.axis_index("c")
        sid = lax.axis_index("s")
        wid = sid * nc + cid
        base = wid * rpw

        pltpu.sync_copy(idx_hbm.at[pl.ds(wid * subs_per_w, subs_per_w)], idx_v)
        pltpu.sync_copy(pe_hbm, pe_v)

        def gather_start(cj, b):
            pltpu.async_copy(
                table_hbm.at[idx_v.at[2 * cj]], bufs[b].at[pl.ds(0, SUB)], sem_g[b])
            pltpu.async_copy(
                table_hbm.at[idx_v.at[2 * cj + 1]], bufs[b].at[pl.ds(SUB, SUB)],
                sem_g[b])

        def gather_wait(cj, b):
            pltpu.make_async_copy(
                table_hbm.at[idx_v.at[2 * cj]], bufs[b].at[pl.ds(0, SUB)],
                sem_g[b]).wait()
            pltpu.make_async_copy(
                table_hbm.at[idx_v.at[2 * cj + 1]], bufs[b].at[pl.ds(SUB, SUB)],
                sem_g[b]).wait()

        def wb_start(cj, b):
            pltpu.async_copy(
                bufs[b], out_hbm.at[pl.ds(base + cj * CHUNK, CHUNK)], sem_w[b])

        def wb_wait(cj, b):
            pltpu.make_async_copy(
                bufs[b], out_hbm.at[pl.ds(base + cj * CHUNK, CHUNK)],
                sem_w[b]).wait()

        def vecpass(b):
            buf = bufs[b]

            def pos_body(pos, carry):
                for l in range(D // 16):
                    sl = pl.ds(16 * l, 16)
                    pv = pe_v[pos, sl]
                    for kk in range(CHUNK // SEQ):
                        r = pos + SEQ * kk
                        buf[r, sl] = buf[r, sl] * 8.0 + pv
                return carry

            lax.fori_loop(0, SEQ, pos_body, 0)

        gather_start(0, 0)

        def round_body(c4, carry):
            for j in range(NBUF):
                cj = NBUF * c4 + j
                nxt = (j + 1) % NBUF
                if j == NBUF - 1:
                    wb_wait(cj - (NBUF - 1), nxt)

                    @pl.when(cj + 1 < n_chunks)
                    def _():
                        gather_start(cj + 1, nxt)
                else:
                    @pl.when(c4 > 0)
                    def _():
                        wb_wait(cj - (NBUF - 1), nxt)

                    gather_start(cj + 1, nxt)
                gather_wait(cj, j)
                vecpass(j)
                wb_start(cj, j)
            return carry

        lax.fori_loop(0, n_rounds, round_body, 0)

        for j in range(1, NBUF):
            wb_wait(n_chunks - NBUF + j, j)

    return k


def kernel(encoded_words, embed_weight, pe):
    batch, seq = encoded_words.shape
    n_rows = batch * seq
    idx = encoded_words.reshape(-1).astype(jnp.int32).reshape(-1, SUB)
    pe2 = pe.reshape(SEQ, D)
    nc, ns = _sc_geometry()
    out = _build_sc_kernel(n_rows, nc, ns)(idx, embed_weight, pe2)
    return out.reshape(batch, seq, D)


# speedup vs baseline: 1.7433x; 1.7433x over previous
import functools
import math

import jax
import jax.numpy as jnp
from jax import lax
from jax.experimental import pallas as pl
from jax.experimental.pallas import tpu as pltpu
from jax.experimental.pallas import tpu_sc as plsc

D = 64
SEQ = 50
SUB = 100
CHUNK = 2 * SUB
NBUF = 4


def _sc_geometry():
    try:
        info = plsc.get_sparse_core_info()
        return info.num_cores, info.num_subcores
    except Exception:
        return 2, 16


def _build_sc_kernel(n_rows, nc, ns):
    nw = nc * ns
    rpw = n_rows // nw
    n_chunks = rpw // CHUNK
    subs_per_w = rpw // SUB
    n_rounds = n_chunks // NBUF

    mesh = plsc.VectorSubcoreMesh(core_axis_name="c", subcore_axis_name="s")

    @functools.partial(
        pl.kernel,
        out_type=jax.ShapeDtypeStruct((n_rows, D), jnp.float32),
        mesh=mesh,
        scratch_types=[
            pltpu.VMEM((subs_per_w, SUB), jnp.int32),
            pltpu.VMEM((SEQ, D), jnp.float32),
        ] + [pltpu.VMEM((CHUNK, D), jnp.float32) for _ in range(NBUF)]
          + [pltpu.SemaphoreType.DMA for _ in range(2 * NBUF)],
        compiler_params=pltpu.CompilerParams(use_tc_tiling_on_sc=False),
    )
    def k(idx_hbm, table_hbm, pe_hbm, out_hbm, idx_v, pe_v, *bufs_and_sems):
        bufs = bufs_and_sems[:NBUF]
        sem_g = bufs_and_sems[NBUF:2 * NBUF]
        sem_w = bufs_and_sems[2 * NBUF:]

        cid = lax.axis_index("c")
        sid = lax.axis_index("s")
        wid = sid * nc + cid
        base = wid * rpw

        pltpu.sync_copy(idx_hbm.at[pl.ds(wid * subs_per_w, subs_per_w)], idx_v)
        pltpu.sync_copy(pe_hbm, pe_v)

        def gather_start(cj, b):
            pltpu.async_copy(
                table_hbm.at[idx_v.at[2 * cj]], bufs[b].at[pl.ds(0, SUB)], sem_g[b])
            pltpu.async_copy(
                table_hbm.at[idx_v.at[2 * cj + 1]], bufs[b].at[pl.ds(SUB, SUB)],
                sem_g[b])

        def gather_wait(cj, b):
            pltpu.make_async_copy(
                table_hbm.at[idx_v.at[2 * cj]], bufs[b].at[pl.ds(0, SUB)],
                sem_g[b]).wait()
            pltpu.make_async_copy(
                table_hbm.at[idx_v.at[2 * cj + 1]], bufs[b].at[pl.ds(SUB, SUB)],
                sem_g[b]).wait()

        def wb_start(cj, b):
            pltpu.async_copy(
                bufs[b], out_hbm.at[pl.ds(base + cj * CHUNK, CHUNK)], sem_w[b])

        def wb_wait(cj, b):
            pltpu.make_async_copy(
                bufs[b], out_hbm.at[pl.ds(base + cj * CHUNK, CHUNK)],
                sem_w[b]).wait()

        def vecpass(b):
            buf = bufs[b]

            def pos_body(pos, carry):
                for l in range(D // 16):
                    sl = pl.ds(16 * l, 16)
                    pv = pe_v[pos, sl]
                    for kk in range(CHUNK // SEQ):
                        r = pos + SEQ * kk
                        buf[r, sl] = buf[r, sl] + pv
                return carry

            lax.fori_loop(0, SEQ, pos_body, 0)

        gather_start(0, 0)

        def round_body(c4, carry):
            for j in range(NBUF):
                cj = NBUF * c4 + j
                nxt = (j + 1) % NBUF
                if j == NBUF - 1:
                    wb_wait(cj - (NBUF - 1), nxt)

                    @pl.when(cj + 1 < n_chunks)
                    def _():
                        gather_start(cj + 1, nxt)
                else:
                    @pl.when(c4 > 0)
                    def _():
                        wb_wait(cj - (NBUF - 1), nxt)

                    gather_start(cj + 1, nxt)
                gather_wait(cj, j)
                vecpass(j)
                wb_start(cj, j)
            return carry

        lax.fori_loop(0, n_rounds, round_body, 0)

        for j in range(1, NBUF):
            wb_wait(n_chunks - NBUF + j, j)

    return k


_T_COLS = 8192


def _build_tc_transpose(v):
    grid = (v + _T_COLS - 1) // _T_COLS
    half = _T_COLS // 2
    scale = float(math.sqrt(D))

    def body(tw_ref, out_ref):
        x = tw_ref[...]
        y = jnp.swapaxes(x, 0, 1)
        out_ref[:, 0:D] = y[0:half, :] * scale
        out_ref[:, D:2 * D] = y[half:_T_COLS, :] * scale

    return grid, pl.pallas_call(
        body,
        grid=(grid,),
        in_specs=[pl.BlockSpec((D, _T_COLS), lambda i: (0, i))],
        out_specs=pl.BlockSpec((half, 2 * D), lambda i: (i, 0)),
        out_shape=jax.ShapeDtypeStruct((grid * half, 2 * D), jnp.float32),
    )


def kernel(encoded_words, embed_weight, pe):
    batch, seq = encoded_words.shape
    n_rows = batch * seq
    v = embed_weight.shape[0]
    pe2 = pe.reshape(SEQ, D)
    nc, ns = _sc_geometry()
    _, transpose_fn = _build_tc_transpose(v)
    t128 = transpose_fn(embed_weight.T)
    table_scaled = t128.reshape(-1, D)
    i = encoded_words.reshape(-1).astype(jnp.int32)
    half = _T_COLS // 2
    j = ((i // _T_COLS) * _T_COLS + (i % half) * 2 + (i % _T_COLS) // half)
    idx = j.reshape(-1, SUB)
    out = _build_sc_kernel(n_rows, nc, ns)(idx, table_scaled, pe2)
    return out.reshape(batch, seq, D)


# speedup vs baseline: 1.8746x; 1.0753x over previous
import functools
import math

import jax
import jax.numpy as jnp
from jax import lax
from jax.experimental import pallas as pl
from jax.experimental.pallas import tpu as pltpu
from jax.experimental.pallas import tpu_sc as plsc

D = 64
SEQ = 50
SUB = 100
CHUNK = 2 * SUB
NBUF = 4


def _sc_geometry():
    try:
        info = plsc.get_sparse_core_info()
        return info.num_cores, info.num_subcores
    except Exception:
        return 2, 16


def _build_sc_kernel(n_rows, nc, ns):
    nw = nc * ns
    rpw = n_rows // nw
    n_chunks = rpw // CHUNK
    subs_per_w = rpw // SUB
    n_rounds = n_chunks // NBUF

    mesh = plsc.VectorSubcoreMesh(core_axis_name="c", subcore_axis_name="s")

    @functools.partial(
        pl.kernel,
        out_type=jax.ShapeDtypeStruct((n_rows, D), jnp.float32),
        mesh=mesh,
        scratch_types=[
            pltpu.VMEM((subs_per_w, SUB), jnp.int32),
            pltpu.VMEM((SEQ, D), jnp.float32),
        ] + [pltpu.VMEM((CHUNK, D), jnp.float32) for _ in range(NBUF)]
          + [pltpu.SemaphoreType.DMA for _ in range(2 * NBUF)],
        compiler_params=pltpu.CompilerParams(use_tc_tiling_on_sc=False),
    )
    def k(idx_hbm, table_hbm, pe_hbm, out_hbm, idx_v, pe_v, *bufs_and_sems):
        bufs = bufs_and_sems[:NBUF]
        sem_g = bufs_and_sems[NBUF:2 * NBUF]
        sem_w = bufs_and_sems[2 * NBUF:]

        cid = lax.axis_index("c")
        sid = lax.axis_index("s")
        wid = sid * nc + cid
        base = wid * rpw

        pltpu.sync_copy(idx_hbm.at[pl.ds(wid * subs_per_w, subs_per_w)], idx_v)
        pltpu.sync_copy(pe_hbm, pe_v)

        def gather_start(cj, b):
            pltpu.async_copy(
                table_hbm.at[idx_v.at[2 * cj]], bufs[b].at[pl.ds(0, SUB)], sem_g[b])
            pltpu.async_copy(
                table_hbm.at[idx_v.at[2 * cj + 1]], bufs[b].at[pl.ds(SUB, SUB)],
                sem_g[b])

        def gather_wait(cj, b):
            pltpu.make_async_copy(
                table_hbm.at[idx_v.at[2 * cj]], bufs[b].at[pl.ds(0, SUB)],
                sem_g[b]).wait()
            pltpu.make_async_copy(
                table_hbm.at[idx_v.at[2 * cj + 1]], bufs[b].at[pl.ds(SUB, SUB)],
                sem_g[b]).wait()

        def wb_start(cj, b):
            pltpu.async_copy(
                bufs[b], out_hbm.at[pl.ds(base + cj * CHUNK, CHUNK)], sem_w[b])

        def wb_wait(cj, b):
            pltpu.make_async_copy(
                bufs[b], out_hbm.at[pl.ds(base + cj * CHUNK, CHUNK)],
                sem_w[b]).wait()

        def vecpass(b):
            buf = bufs[b]

            def pos_body(pos, carry):
                for l in range(D // 16):
                    sl = pl.ds(16 * l, 16)
                    pv = pe_v[pos, sl]
                    for kk in range(CHUNK // SEQ):
                        r = pos + SEQ * kk
                        buf[r, sl] = buf[r, sl] + pv
                return carry

            lax.fori_loop(0, SEQ, pos_body, 0)

        gather_start(0, 0)

        def round_body(c4, carry):
            for j in range(NBUF):
                cj = NBUF * c4 + j
                nxt = (j + 1) % NBUF
                if j == NBUF - 1:
                    wb_wait(cj - (NBUF - 1), nxt)

                    @pl.when(cj + 1 < n_chunks)
                    def _():
                        gather_start(cj + 1, nxt)
                else:
                    @pl.when(c4 > 0)
                    def _():
                        wb_wait(cj - (NBUF - 1), nxt)

                    gather_start(cj + 1, nxt)
                gather_wait(cj, j)
                vecpass(j)
                wb_start(cj, j)
            return carry

        lax.fori_loop(0, n_rounds, round_body, 0)

        for j in range(1, NBUF):
            wb_wait(n_chunks - NBUF + j, j)

    return k


_T_COLS = 16384


def _build_tc_transpose(v):
    grid = (v + _T_COLS - 1) // _T_COLS
    half = _T_COLS // 2
    scale = float(math.sqrt(D))

    def body(tw_ref, out_ref):
        x = tw_ref[...]
        y = jnp.swapaxes(x, 0, 1)
        out_ref[:, 0:D] = y[0:half, :] * scale
        out_ref[:, D:2 * D] = y[half:_T_COLS, :] * scale

    return grid, pl.pallas_call(
        body,
        grid=(grid,),
        in_specs=[pl.BlockSpec((D, _T_COLS), lambda i: (0, i))],
        out_specs=pl.BlockSpec((half, 2 * D), lambda i: (i, 0)),
        out_shape=jax.ShapeDtypeStruct((grid * half, 2 * D), jnp.float32),
    )


def kernel(encoded_words, embed_weight, pe):
    batch, seq = encoded_words.shape
    n_rows = batch * seq
    v = embed_weight.shape[0]
    pe2 = pe.reshape(SEQ, D)
    nc, ns = _sc_geometry()
    _, transpose_fn = _build_tc_transpose(v)
    t128 = transpose_fn(embed_weight.T)
    table_scaled = t128.reshape(-1, D)
    i = encoded_words.reshape(-1).astype(jnp.int32)
    half = _T_COLS // 2
    j = ((i // _T_COLS) * _T_COLS + (i % half) * 2 + (i % _T_COLS) // half)
    idx = j.reshape(-1, SUB)
    out = _build_sc_kernel(n_rows, nc, ns)(idx, table_scaled, pe2)
    return out.reshape(batch, seq, D)
